# baseline (speedup 1.0000x reference)
import functools

import jax
import jax.numpy as jnp
from jax import lax
from jax.experimental import pallas as pl
from jax.experimental.pallas import tpu as pltpu
from jax.experimental.pallas import tpu_sc as plsc

N_NODES = 10000
HIDDEN = 128
OUT_DIM = 1
NUM_GRAPHS = 256

NC = 2
NS = 16
NW = NC * NS
RPW = 320
NCHUNK = 3
IDX_PER_W = NCHUNK * 128
DUMMY = NUM_GRAPHS
ACC_ROWS = 384
ZROWS = ACC_ROWS // NS
LAST_W = NW - 1
LAST_ROWS = N_NODES - LAST_W * RPW


def _sc_segment_sum(h_v, idx3):
    mesh = plsc.VectorSubcoreMesh(core_axis_name="c", subcore_axis_name="s",
                                  num_cores=NC, num_subcores=NS)

    @functools.partial(
        pl.kernel,
        out_type=jax.ShapeDtypeStruct((NC, NUM_GRAPHS, HIDDEN), jnp.float32),
        mesh=mesh,
        scratch_types=[
            pltpu.VMEM((IDX_PER_W, HIDDEN), jnp.float32),
            pltpu.VMEM((NCHUNK, 128), jnp.int32),
            pltpu.VMEM((ZROWS, HIDDEN), jnp.float32),
            pltpu.VMEM_SHARED((ACC_ROWS, HIDDEN), jnp.float32),
            pltpu.SemaphoreType.DMA,
            pltpu.SemaphoreType.DMA,
        ],
    )
    def seg_sum(h_hbm, idx_hbm, sums_out,
                hbuf, idxbuf, zbuf, acc,
                hsem, isem):
        c = lax.axis_index("c")
        s = lax.axis_index("s")
        w = c * NS + s

        idx_cp = pltpu.async_copy(idx_hbm.at[w], idxbuf, isem)

        @pl.when(w < LAST_W)
        def _():
            pltpu.async_copy(h_hbm.at[pl.ds(w * RPW, RPW)],
                             hbuf.at[pl.ds(0, RPW)], hsem).wait()

        @pl.when(w == LAST_W)
        def _():
            pltpu.async_copy(h_hbm.at[pl.ds(LAST_W * RPW, LAST_ROWS)],
                             hbuf.at[pl.ds(0, LAST_ROWS)], hsem).wait()

        zero16 = jnp.zeros((16,), jnp.float32)
        for r in range(ZROWS):
            for j in range(HIDDEN // 16):
                zbuf[r, pl.ds(j * 16, 16)] = zero16

        pltpu.sync_copy(zbuf, acc.at[pl.ds(s * ZROWS, ZROWS)])
        idx_cp.wait()
        plsc.subcore_barrier()

        for j in range(NCHUNK):
            pltpu.sync_copy(hbuf.at[pl.ds(j * 128, 128)],
                            acc.at[idxbuf.at[j]], add=True)
        plsc.subcore_barrier()

        pltpu.sync_copy(acc.at[pl.ds(s * 16, 16)],
                        sums_out.at[c, pl.ds(s * 16, 16)])

    return seg_sum(h_v, idx3)


def _mlp_kernel(s_ref, b_ref, w1_ref, b1_ref, w2_ref, b2_ref, w3_ref, b3_ref,
                out_ref):
    sums = s_ref[0] + s_ref[1]
    seg = lax.broadcasted_iota(jnp.int32, (NUM_GRAPHS, 1), 0)
    onehot = (b_ref[...] == seg).astype(jnp.float32)
    counts = jnp.sum(onehot, axis=1, keepdims=True)
    pooled = sums / jnp.maximum(counts, 1.0)
    x = jnp.maximum(
        jnp.dot(pooled, w1_ref[...], preferred_element_type=jnp.float32)
        + b1_ref[...], 0.0)
    x = jnp.maximum(
        jnp.dot(x, w2_ref[...], preferred_element_type=jnp.float32)
        + b2_ref[...], 0.0)
    pred = jnp.dot(x, w3_ref[...], preferred_element_type=jnp.float32) \
        + b3_ref[...]
    out_ref[...] = jnp.where(counts > 0.0, pred, 0.0)


def kernel(h_v, edge_index, batch, W1, b1, W2, b2, W3, b3):
    del edge_index
    b32 = batch.astype(jnp.int32)
    bp = jnp.full((NW * RPW,), DUMMY, jnp.int32).at[:N_NODES].set(b32)
    idx3 = (jnp.full((NW, IDX_PER_W), DUMMY, jnp.int32)
            .at[:, :RPW].set(bp.reshape(NW, RPW))
            .reshape(NW, NCHUNK, 128))
    sums = _sc_segment_sum(h_v, idx3)
    return pl.pallas_call(
        _mlp_kernel,
        out_shape=jax.ShapeDtypeStruct((NUM_GRAPHS, OUT_DIM), jnp.float32),
    )(sums, b32.reshape(1, N_NODES), W1, b1.reshape(1, HIDDEN),
      W2, b2.reshape(1, HIDDEN), W3, b3.reshape(1, OUT_DIM))
